# baseline (speedup 1.0000x reference)
import functools

import jax
import jax.numpy as jnp
from jax import lax
from jax.experimental import pallas as pl
from jax.experimental.pallas import tpu as pltpu

_K = 10
_ALPHA = 0.1
_TILE = 10240


def _make_body(n):
    num_tiles = pl.cdiv(n, _TILE)
    edge = n - (num_tiles - 1) * _TILE

    def _body(x_ref, ut_ref, v_ref, w_ref, out_ref, s_ref, g_ref, wa_ref):
        phase = pl.program_id(0)
        i = pl.program_id(1)

        @pl.when((phase == 0) & (i == 0))
        def _init():
            s_ref[...] = jnp.zeros_like(s_ref)

        @pl.when((phase == 0) & (i < num_tiles - 1))
        def _accum_full():
            s_ref[...] += lax.dot_general(
                ut_ref[...].astype(jnp.bfloat16),
                x_ref[...].astype(jnp.bfloat16),
                (((1,), (0,)), ((), ())),
                preferred_element_type=jnp.float32,
            )

        @pl.when((phase == 0) & (i == num_tiles - 1))
        def _accum_edge_and_small_mats():
            s = s_ref[...] + lax.dot_general(
                ut_ref[:, :edge].astype(jnp.bfloat16),
                x_ref[:edge, :].astype(jnp.bfloat16),
                (((1,), (0,)), ((), ())),
                preferred_element_type=jnp.float32,
            )

            v = v_ref[...]
            v_pow = jnp.ones_like(v)
            v_out = jnp.zeros_like(v)
            for _ in range(_K):
                v_pow = v_pow * v
                v_out = v_out + (1.0 - _ALPHA) * v_pow
            v_out = v_out / _K

            g32 = lax.dot_general(
                v_out * s, w_ref[...],
                (((1,), (1,)), ((), ())),
                preferred_element_type=jnp.float32,
            )
            g_ref[...] = g32.astype(jnp.bfloat16)
            wa_ref[...] = (_ALPHA * w_ref[...]).astype(jnp.bfloat16)

        @pl.when(phase == 1)
        def _emit():
            out_ref[...] = lax.dot_general(
                ut_ref[...].astype(jnp.bfloat16), g_ref[...],
                (((0,), (0,)), ((), ())),
                preferred_element_type=jnp.float32,
            ) + lax.dot_general(
                x_ref[...].astype(jnp.bfloat16), wa_ref[...],
                (((1,), (1,)), ((), ())),
                preferred_element_type=jnp.float32,
            )

    return _body


@functools.partial(jax.jit, static_argnames=())
def kernel(x, U, V, W):
    n, d = x.shape
    keig = U.shape[1]
    num_tiles = pl.cdiv(n, _TILE)

    v2 = V.reshape(keig, 1)
    ut = U.T

    out = pl.pallas_call(
        _make_body(n),
        grid=(2, num_tiles),
        in_specs=[
            pl.BlockSpec((_TILE, d), lambda p, i: (i, 0)),
            pl.BlockSpec((keig, _TILE), lambda p, i: (0, i)),
            pl.BlockSpec((keig, 1), lambda p, i: (0, 0)),
            pl.BlockSpec((d, d), lambda p, i: (0, 0)),
        ],
        out_specs=pl.BlockSpec((_TILE, d), lambda p, i: (p * i, 0)),
        out_shape=jax.ShapeDtypeStruct((n, d), jnp.float32),
        scratch_shapes=[
            pltpu.VMEM((keig, d), jnp.float32),
            pltpu.VMEM((keig, d), jnp.bfloat16),
            pltpu.VMEM((d, d), jnp.bfloat16),
        ],
        compiler_params=pltpu.CompilerParams(
            dimension_semantics=("arbitrary", "arbitrary"),
        ),
    )(x, ut, v2, W)
    return out


# speedup vs baseline: 1.0018x; 1.0018x over previous
import functools

import jax
import jax.numpy as jnp
from jax import lax
from jax.experimental import pallas as pl
from jax.experimental.pallas import tpu as pltpu

_K = 10
_ALPHA = 0.1
_TILE = 12800


def _make_body(n):
    num_tiles = pl.cdiv(n, _TILE)
    edge = n - (num_tiles - 1) * _TILE

    def _body(x_ref, ut_ref, v_ref, w_ref, out_ref, s_ref, g_ref, wa_ref):
        phase = pl.program_id(0)
        i = pl.program_id(1)

        @pl.when((phase == 0) & (i == 0))
        def _init():
            s_ref[...] = jnp.zeros_like(s_ref)

        @pl.when((phase == 0) & (i < num_tiles - 1))
        def _accum_full():
            s_ref[...] += lax.dot_general(
                ut_ref[...].astype(jnp.bfloat16),
                x_ref[...].astype(jnp.bfloat16),
                (((1,), (0,)), ((), ())),
                preferred_element_type=jnp.float32,
            )

        @pl.when((phase == 0) & (i == num_tiles - 1))
        def _accum_edge_and_small_mats():
            s = s_ref[...] + lax.dot_general(
                ut_ref[:, :edge].astype(jnp.bfloat16),
                x_ref[:edge, :].astype(jnp.bfloat16),
                (((1,), (0,)), ((), ())),
                preferred_element_type=jnp.float32,
            )

            v = v_ref[...]
            v_pow = jnp.ones_like(v)
            v_out = jnp.zeros_like(v)
            for _ in range(_K):
                v_pow = v_pow * v
                v_out = v_out + (1.0 - _ALPHA) * v_pow
            v_out = v_out / _K

            g32 = lax.dot_general(
                v_out * s, w_ref[...],
                (((1,), (1,)), ((), ())),
                preferred_element_type=jnp.float32,
            )
            g_ref[...] = g32.astype(jnp.bfloat16)
            wa_ref[...] = (_ALPHA * w_ref[...]).astype(jnp.bfloat16)

        @pl.when(phase == 1)
        def _emit():
            out_ref[...] = lax.dot_general(
                ut_ref[...].astype(jnp.bfloat16), g_ref[...],
                (((0,), (0,)), ((), ())),
                preferred_element_type=jnp.float32,
            ) + lax.dot_general(
                x_ref[...].astype(jnp.bfloat16), wa_ref[...],
                (((1,), (1,)), ((), ())),
                preferred_element_type=jnp.float32,
            )

    return _body


@functools.partial(jax.jit, static_argnames=())
def kernel(x, U, V, W):
    n, d = x.shape
    keig = U.shape[1]
    num_tiles = pl.cdiv(n, _TILE)

    v2 = V.reshape(keig, 1)
    ut = U.T

    out = pl.pallas_call(
        _make_body(n),
        grid=(2, num_tiles),
        in_specs=[
            pl.BlockSpec((_TILE, d), lambda p, i: (i, 0)),
            pl.BlockSpec((keig, _TILE), lambda p, i: (0, i)),
            pl.BlockSpec((keig, 1), lambda p, i: (0, 0)),
            pl.BlockSpec((d, d), lambda p, i: (0, 0)),
        ],
        out_specs=pl.BlockSpec((_TILE, d), lambda p, i: (p * i, 0)),
        out_shape=jax.ShapeDtypeStruct((n, d), jnp.float32),
        scratch_shapes=[
            pltpu.VMEM((keig, d), jnp.float32),
            pltpu.VMEM((keig, d), jnp.bfloat16),
            pltpu.VMEM((d, d), jnp.bfloat16),
        ],
        compiler_params=pltpu.CompilerParams(
            dimension_semantics=("arbitrary", "arbitrary"),
        ),
    )(x, ut, v2, W)
    return out


# speedup vs baseline: 1.1887x; 1.1865x over previous
import functools

import jax
import jax.numpy as jnp
from jax import lax
from jax.experimental import pallas as pl
from jax.experimental.pallas import tpu as pltpu

_K = 10
_ALPHA = 0.1
_TILE = 12800
_RETAIN = 4


def _make_body(n):
    num_tiles = pl.cdiv(n, _TILE)
    edge = n - (num_tiles - 1) * _TILE
    n_stream = num_tiles - _RETAIN

    def _body(x_ref, ut_ref, v_ref, w_ref, out_ref,
              s_ref, g_ref, wa_ref, xs_ref, us_ref):
        phase = pl.program_id(0)
        i = pl.program_id(1)

        @pl.when((phase == 0) & (i == 0))
        def _init():
            s_ref[...] = jnp.zeros_like(s_ref)

        for r in range(_RETAIN):
            @pl.when((phase == 0) & (i == r))
            def _accum_retain(r=r):
                xb = x_ref[...].astype(jnp.bfloat16)
                ub = ut_ref[...].astype(jnp.bfloat16)
                xs_ref[r * _TILE:(r + 1) * _TILE, :] = xb
                us_ref[:, r * _TILE:(r + 1) * _TILE] = ub
                s_ref[...] += lax.dot_general(
                    ub, xb,
                    (((1,), (0,)), ((), ())),
                    preferred_element_type=jnp.float32,
                )

        @pl.when((phase == 0) & (i >= _RETAIN) & (i < num_tiles - 1))
        def _accum_full():
            s_ref[...] += lax.dot_general(
                ut_ref[...].astype(jnp.bfloat16),
                x_ref[...].astype(jnp.bfloat16),
                (((1,), (0,)), ((), ())),
                preferred_element_type=jnp.float32,
            )

        @pl.when((phase == 0) & (i == num_tiles - 1))
        def _accum_edge_and_small_mats():
            s = s_ref[...] + lax.dot_general(
                ut_ref[:, :edge].astype(jnp.bfloat16),
                x_ref[:edge, :].astype(jnp.bfloat16),
                (((1,), (0,)), ((), ())),
                preferred_element_type=jnp.float32,
            )

            v = v_ref[...]
            v_pow = jnp.ones_like(v)
            v_out = jnp.zeros_like(v)
            for _ in range(_K):
                v_pow = v_pow * v
                v_out = v_out + (1.0 - _ALPHA) * v_pow
            v_out = v_out / _K

            g32 = lax.dot_general(
                v_out * s, w_ref[...],
                (((1,), (1,)), ((), ())),
                preferred_element_type=jnp.float32,
            )
            g_ref[...] = g32.astype(jnp.bfloat16)
            wa_ref[...] = (_ALPHA * w_ref[...]).astype(jnp.bfloat16)

        def _emit(ub, xb):
            out_ref[...] = lax.dot_general(
                ub, g_ref[...],
                (((0,), (0,)), ((), ())),
                preferred_element_type=jnp.float32,
            ) + lax.dot_general(
                xb, wa_ref[...],
                (((1,), (1,)), ((), ())),
                preferred_element_type=jnp.float32,
            )

        @pl.when((phase == 1) & (i < n_stream))
        def _emit_streamed():
            _emit(ut_ref[...].astype(jnp.bfloat16),
                  x_ref[...].astype(jnp.bfloat16))

        for r in range(_RETAIN):
            @pl.when((phase == 1) & (i == n_stream + r))
            def _emit_retained(r=r):
                _emit(us_ref[:, r * _TILE:(r + 1) * _TILE],
                      xs_ref[r * _TILE:(r + 1) * _TILE, :])

    return _body


@functools.partial(jax.jit, static_argnames=())
def kernel(x, U, V, W):
    n, d = x.shape
    keig = U.shape[1]
    num_tiles = pl.cdiv(n, _TILE)
    n_stream = num_tiles - _RETAIN

    v2 = V.reshape(keig, 1)
    ut = U.T

    def _in_idx(p, i):
        streamed = jnp.minimum(i + _RETAIN, num_tiles - 1)
        return jnp.where(p == 0, i, streamed)

    def _out_idx(p, i):
        t = jnp.where(i < n_stream, i + _RETAIN, i - n_stream)
        return jnp.where(p == 0, _RETAIN, t)

    out = pl.pallas_call(
        _make_body(n),
        grid=(2, num_tiles),
        in_specs=[
            pl.BlockSpec((_TILE, d), lambda p, i: (_in_idx(p, i), 0)),
            pl.BlockSpec((keig, _TILE), lambda p, i: (0, _in_idx(p, i))),
            pl.BlockSpec((keig, 1), lambda p, i: (0, 0)),
            pl.BlockSpec((d, d), lambda p, i: (0, 0)),
        ],
        out_specs=pl.BlockSpec((_TILE, d), lambda p, i: (_out_idx(p, i), 0)),
        out_shape=jax.ShapeDtypeStruct((n, d), jnp.float32),
        scratch_shapes=[
            pltpu.VMEM((keig, d), jnp.float32),
            pltpu.VMEM((keig, d), jnp.bfloat16),
            pltpu.VMEM((d, d), jnp.bfloat16),
            pltpu.VMEM((_RETAIN * _TILE, d), jnp.bfloat16),
            pltpu.VMEM((keig, _RETAIN * _TILE), jnp.bfloat16),
        ],
        compiler_params=pltpu.CompilerParams(
            dimension_semantics=("arbitrary", "arbitrary"),
        ),
    )(x, ut, v2, W)
    return out


# speedup vs baseline: 1.2953x; 1.0897x over previous
import functools

import jax
import jax.numpy as jnp
from jax import lax
from jax.experimental import pallas as pl
from jax.experimental.pallas import tpu as pltpu

_K = 10
_ALPHA = 0.1
_TILE = 6400


def _make_body(n):
    num_tiles = pl.cdiv(n, _TILE)
    edge = n - (num_tiles - 1) * _TILE

    def _body(x_ref, ut_ref, v_ref, w_ref, out_ref,
              s_ref, g_ref, wa_ref, xs_ref, us_ref):
        phase = pl.program_id(0)
        i = pl.program_id(1)

        @pl.when((phase == 0) & (i == 0))
        def _init():
            s_ref[...] = jnp.zeros_like(s_ref)

        for r in range(num_tiles):
            @pl.when((phase == 0) & (i == r))
            def _accum_and_stash(r=r):
                xb = x_ref[...].astype(jnp.bfloat16)
                ub = ut_ref[...].astype(jnp.bfloat16)
                xs_ref[r * _TILE:(r + 1) * _TILE, :] = xb
                us_ref[:, r * _TILE:(r + 1) * _TILE] = ub
                if r < num_tiles - 1:
                    s_ref[...] += lax.dot_general(
                        ub, xb,
                        (((1,), (0,)), ((), ())),
                        preferred_element_type=jnp.float32,
                    )
                else:
                    s = s_ref[...] + lax.dot_general(
                        ub[:, :edge], xb[:edge, :],
                        (((1,), (0,)), ((), ())),
                        preferred_element_type=jnp.float32,
                    )

                    v = v_ref[...]
                    v_pow = jnp.ones_like(v)
                    v_out = jnp.zeros_like(v)
                    for _ in range(_K):
                        v_pow = v_pow * v
                        v_out = v_out + (1.0 - _ALPHA) * v_pow
                    v_out = v_out / _K

                    g32 = lax.dot_general(
                        v_out * s, w_ref[...],
                        (((1,), (1,)), ((), ())),
                        preferred_element_type=jnp.float32,
                    )
                    g_ref[...] = g32.astype(jnp.bfloat16)
                    wa_ref[...] = (_ALPHA * w_ref[...]).astype(jnp.bfloat16)

        for r in range(num_tiles):
            @pl.when((phase == 1) & (i == r))
            def _emit(r=r):
                out_ref[...] = lax.dot_general(
                    us_ref[:, r * _TILE:(r + 1) * _TILE], g_ref[...],
                    (((0,), (0,)), ((), ())),
                    preferred_element_type=jnp.float32,
                ) + lax.dot_general(
                    xs_ref[r * _TILE:(r + 1) * _TILE, :], wa_ref[...],
                    (((1,), (1,)), ((), ())),
                    preferred_element_type=jnp.float32,
                )

    return _body


@functools.partial(jax.jit, static_argnames=())
def kernel(x, U, V, W):
    n, d = x.shape
    keig = U.shape[1]
    num_tiles = pl.cdiv(n, _TILE)

    v2 = V.reshape(keig, 1)
    ut = U.T

    def _in_idx(p, i):
        return jnp.where(p == 0, i, num_tiles - 1)

    out = pl.pallas_call(
        _make_body(n),
        grid=(2, num_tiles),
        in_specs=[
            pl.BlockSpec((_TILE, d), lambda p, i: (_in_idx(p, i), 0)),
            pl.BlockSpec((keig, _TILE), lambda p, i: (0, _in_idx(p, i))),
            pl.BlockSpec((keig, 1), lambda p, i: (0, 0)),
            pl.BlockSpec((d, d), lambda p, i: (0, 0)),
        ],
        out_specs=pl.BlockSpec(
            (_TILE, d), lambda p, i: (jnp.where(p == 0, 0, i), 0)
        ),
        out_shape=jax.ShapeDtypeStruct((n, d), jnp.float32),
        scratch_shapes=[
            pltpu.VMEM((keig, d), jnp.float32),
            pltpu.VMEM((keig, d), jnp.bfloat16),
            pltpu.VMEM((d, d), jnp.bfloat16),
            pltpu.VMEM((num_tiles * _TILE, d), jnp.bfloat16),
            pltpu.VMEM((keig, num_tiles * _TILE), jnp.bfloat16),
        ],
        compiler_params=pltpu.CompilerParams(
            dimension_semantics=("arbitrary", "arbitrary"),
        ),
    )(x, ut, v2, W)
    return out
